# baseline (speedup 1.0000x reference)
import functools

import jax
import jax.numpy as jnp
from jax import lax
from jax.experimental import pallas as pl
from jax.experimental.pallas import tpu as pltpu
from jax.experimental.pallas import tpu_sc as plsc


_HB = 4096


def _transpose_pack_body(xe_ref, xo_ref, y_ref):
    y_ref[...] = jnp.concatenate(
        [jnp.swapaxes(xe_ref[...], 0, 1), jnp.swapaxes(xo_ref[...], 0, 1)],
        axis=1,
    )


def _transpose_pack(table_t):
    D, V = table_t.shape
    grid = (V + 2 * _HB - 1) // (2 * _HB)
    return pl.pallas_call(
        _transpose_pack_body,
        grid=(grid,),
        in_specs=[
            pl.BlockSpec((D, _HB), lambda i: (0, 2 * i)),
            pl.BlockSpec(
                (D, _HB),
                lambda i: (0, jnp.minimum(2 * i + 1, V // _HB - 1)),
            ),
        ],
        out_specs=pl.BlockSpec((_HB, 2 * D), lambda i: (i, 0)),
        out_shape=jax.ShapeDtypeStruct((grid * _HB, 2 * D), jnp.float32),
    )(table_t, table_t)


def kernel(inputs, table):
    B, S = inputs.shape
    V, D = table.shape
    N = B * S
    D2 = 2 * D

    tpack = _transpose_pack(jnp.swapaxes(table, 0, 1))

    info = plsc.get_sparse_core_info()
    NC, NS, L = info.num_cores, info.num_subcores, info.num_lanes
    NW = NC * NS
    per_w = N // NW
    G = 128
    K = 2
    C = G * K
    n_chunks = per_w // C

    idx_flat = inputs.reshape(N)
    mesh = plsc.VectorSubcoreMesh(core_axis_name="c", subcore_axis_name="s")

    @functools.partial(
        pl.kernel,
        mesh=mesh,
        compiler_params=pltpu.CompilerParams(
            use_tc_tiling_on_sc=False, needs_layout_passes=False
        ),
        out_type=jax.ShapeDtypeStruct((N // 2, D2), jnp.float32),
        scratch_types=[
            pltpu.VMEM((per_w,), jnp.int32),
            pltpu.VMEM((per_w,), jnp.int32),
            pltpu.VMEM((2, C, D2), jnp.float32),
            pltpu.VMEM((2, C // 2, D2), jnp.float32),
            pltpu.SemaphoreType.DMA,
            pltpu.SemaphoreType.DMA,
            pltpu.SemaphoreType.DMA,
            pltpu.SemaphoreType.DMA,
        ],
    )
    def emb(idx_hbm, tab_hbm, out_hbm, idx_v, idx2_v, slab_v, outb_v,
            g0, g1, o0, o1):
        wid = lax.axis_index("s") * NC + lax.axis_index("c")
        base = wid * per_w
        pltpu.sync_copy(idx_hbm.at[pl.ds(base, per_w)], idx_v)

        hb_bits = _HB.bit_length() - 1

        def halve(i, carry):
            v = idx_v[pl.ds(i * L, L)]
            hi = jax.lax.shift_right_logical(v, hb_bits + 1)
            lo = jax.lax.bitwise_and(v, _HB - 1)
            idx2_v[pl.ds(i * L, L)] = jax.lax.shift_left(hi, hb_bits) + lo
            return carry

        lax.fori_loop(0, per_w // L, halve, None)

        gsem = (g0, g1)
        osem = (o0, o1)
        gd = {0: [], 1: []}
        od = {0: None, 1: None}

        def fire(g):
            b = g % 2
            gd[b] = [
                pltpu.async_copy(
                    tab_hbm.at[idx2_v.at[pl.ds(g * C + k * G, G)]],
                    slab_v.at[b, pl.ds(k * G, G)],
                    gsem[b],
                )
                for k in range(K)
            ]

        def select(g):
            b = g % 2
            slab = slab_v.at[b]
            outb = outb_v.at[b]

            def grp(j, carry):
                rows = j * L + lax.broadcasted_iota(jnp.int32, (L,), 0)
                par = jax.lax.bitwise_and(
                    jax.lax.shift_right_logical(
                        idx_v[pl.ds(g * C + j * L, L)], hb_bits
                    ),
                    1,
                )
                src_base = par * D
                dst_row = jax.lax.shift_right_logical(rows, 1)
                dst_base = jax.lax.bitwise_and(rows, 1) * D

                def dcol(d, carry2):
                    vals = plsc.load_gather(slab, [rows, src_base + d])
                    plsc.store_scatter(outb, [dst_row, dst_base + d], vals)
                    return carry2

                lax.fori_loop(0, D, dcol, None)
                return carry

            lax.fori_loop(0, C // L, grp, None)

        fire(0)
        for g in range(n_chunks):
            b = g % 2
            if g + 1 < n_chunks:
                if od[1 - b] is not None:
                    od[1 - b].wait()
                    od[1 - b] = None
                fire(g + 1)
            for d in gd[b]:
                d.wait()
            select(g)
            od[b] = pltpu.async_copy(
                outb_v.at[b],
                out_hbm.at[pl.ds(pl.multiple_of((base + g * C) // 2, 8), C // 2)],
                osem[b],
            )
        for b in (0, 1):
            if od[b] is not None:
                od[b].wait()

    out = emb(idx_flat, tpack)
    return out.reshape(B, S, D)


# speedup vs baseline: 2.0106x; 2.0106x over previous
import functools

import jax
import jax.numpy as jnp
from jax import lax
from jax.experimental import pallas as pl
from jax.experimental.pallas import tpu as pltpu
from jax.experimental.pallas import tpu_sc as plsc


_HB = 4096


def _transpose_pack_body(xe_ref, xo_ref, y_ref):
    y_ref[...] = jnp.concatenate(
        [jnp.swapaxes(xe_ref[...], 0, 1), jnp.swapaxes(xo_ref[...], 0, 1)],
        axis=1,
    )


def _transpose_pack(table_t):
    D, V = table_t.shape
    grid = (V + 2 * _HB - 1) // (2 * _HB)
    return pl.pallas_call(
        _transpose_pack_body,
        grid=(grid,),
        in_specs=[
            pl.BlockSpec((D, _HB), lambda i: (0, 2 * i)),
            pl.BlockSpec(
                (D, _HB),
                lambda i: (0, jnp.minimum(2 * i + 1, V // _HB - 1)),
            ),
        ],
        out_specs=pl.BlockSpec((_HB, 2 * D), lambda i: (i, 0)),
        out_shape=jax.ShapeDtypeStruct((grid * _HB, 2 * D), jnp.float32),
    )(table_t, table_t)


def kernel(inputs, table):
    B, S = inputs.shape
    V, D = table.shape
    N = B * S

    tpack = _transpose_pack(jnp.swapaxes(table, 0, 1))
    t64 = tpack.reshape(2 * tpack.shape[0], D)

    info = plsc.get_sparse_core_info()
    NC, NS, L = info.num_cores, info.num_subcores, info.num_lanes
    NW = NC * NS
    per_w = N // NW
    G = 128
    K = 5
    C = G * K
    n_chunks = per_w // C

    idx_flat = inputs.reshape(N)
    mesh = plsc.VectorSubcoreMesh(core_axis_name="c", subcore_axis_name="s")

    @functools.partial(
        pl.kernel,
        mesh=mesh,
        compiler_params=pltpu.CompilerParams(
            use_tc_tiling_on_sc=False, needs_layout_passes=False
        ),
        out_type=jax.ShapeDtypeStruct((N, D), jnp.float32),
        scratch_types=[
            pltpu.VMEM((per_w,), jnp.int32),
            pltpu.VMEM((per_w,), jnp.int32),
            pltpu.VMEM((2, C, D), jnp.float32),
            pltpu.SemaphoreType.DMA,
            pltpu.SemaphoreType.DMA,
            pltpu.SemaphoreType.DMA,
            pltpu.SemaphoreType.DMA,
        ],
    )
    def emb(idx_hbm, tab_hbm, out_hbm, idx_v, idx2_v, rows_v, g0, g1, o0, o1):
        wid = lax.axis_index("s") * NC + lax.axis_index("c")
        base = wid * per_w
        pltpu.sync_copy(idx_hbm.at[pl.ds(base, per_w)], idx_v)

        hb_bits = _HB.bit_length() - 1

        def xform(i, carry):
            v = idx_v[pl.ds(i * L, L)]
            hi = jax.lax.shift_left(
                jax.lax.shift_right_logical(v, hb_bits + 1), hb_bits + 1
            )
            lo = jax.lax.shift_left(jax.lax.bitwise_and(v, _HB - 1), 1)
            b1 = jax.lax.bitwise_and(
                jax.lax.shift_right_logical(v, hb_bits), 1
            )
            idx2_v[pl.ds(i * L, L)] = hi + lo + b1
            return carry

        lax.fori_loop(0, per_w // L, xform, None)

        gsem = (g0, g1)
        osem = (o0, o1)
        gd = {0: [], 1: []}
        od = {0: None, 1: None}

        def fire(g):
            b = g % 2
            gd[b] = [
                pltpu.async_copy(
                    tab_hbm.at[idx2_v.at[pl.ds(g * C + k * G, G)]],
                    rows_v.at[b, pl.ds(k * G, G)],
                    gsem[b],
                )
                for k in range(K)
            ]

        fire(0)
        for g in range(n_chunks):
            b = g % 2
            if g + 1 < n_chunks:
                if od[1 - b] is not None:
                    od[1 - b].wait()
                    od[1 - b] = None
                fire(g + 1)
            for d in gd[b]:
                d.wait()
            od[b] = pltpu.async_copy(
                rows_v.at[b], out_hbm.at[pl.ds(base + g * C, C)], osem[b]
            )
        for b in (0, 1):
            if od[b] is not None:
                od[b].wait()

    out = emb(idx_flat, t64)
    return out.reshape(B, S, D)


# speedup vs baseline: 2.1795x; 1.0840x over previous
import functools

import jax
import jax.numpy as jnp
from jax import lax
from jax.experimental import pallas as pl
from jax.experimental.pallas import tpu as pltpu
from jax.experimental.pallas import tpu_sc as plsc


_HB = 8192


def _transpose_pack_body(xe_ref, xo_ref, y_ref):
    d = xe_ref.shape[0]
    y_ref[:, 0:d] = jnp.swapaxes(xe_ref[...], 0, 1)
    y_ref[:, d : 2 * d] = jnp.swapaxes(xo_ref[...], 0, 1)


def _transpose_pack(table_t):
    D, V = table_t.shape
    grid = (V + 2 * _HB - 1) // (2 * _HB)
    return pl.pallas_call(
        _transpose_pack_body,
        grid=(grid,),
        in_specs=[
            pl.BlockSpec((D, _HB), lambda i: (0, 2 * i)),
            pl.BlockSpec(
                (D, _HB),
                lambda i: (0, jnp.minimum(2 * i + 1, V // _HB - 1)),
            ),
        ],
        out_specs=pl.BlockSpec((_HB, 2 * D), lambda i: (i, 0)),
        out_shape=jax.ShapeDtypeStruct((grid * _HB, 2 * D), jnp.float32),
    )(table_t, table_t)


def kernel(inputs, table):
    B, S = inputs.shape
    V, D = table.shape
    N = B * S

    tpack = _transpose_pack(jnp.swapaxes(table, 0, 1))
    t64 = tpack.reshape(2 * tpack.shape[0], D)

    info = plsc.get_sparse_core_info()
    NC, NS, L = info.num_cores, info.num_subcores, info.num_lanes
    NW = NC * NS
    per_w = N // NW
    G = 128
    K = 5
    C = G * K
    n_chunks = per_w // C

    idx_flat = inputs.reshape(N)
    mesh = plsc.VectorSubcoreMesh(core_axis_name="c", subcore_axis_name="s")

    @functools.partial(
        pl.kernel,
        mesh=mesh,
        compiler_params=pltpu.CompilerParams(
            use_tc_tiling_on_sc=False, needs_layout_passes=False
        ),
        out_type=jax.ShapeDtypeStruct((N, D), jnp.float32),
        scratch_types=[
            pltpu.VMEM((per_w,), jnp.int32),
            pltpu.VMEM((per_w,), jnp.int32),
            pltpu.VMEM((2, C, D), jnp.float32),
            pltpu.SemaphoreType.DMA,
            pltpu.SemaphoreType.DMA,
            pltpu.SemaphoreType.DMA,
            pltpu.SemaphoreType.DMA,
        ],
    )
    def emb(idx_hbm, tab_hbm, out_hbm, idx_v, idx2_v, rows_v, g0, g1, o0, o1):
        wid = lax.axis_index("s") * NC + lax.axis_index("c")
        base = wid * per_w
        pltpu.sync_copy(idx_hbm.at[pl.ds(base, per_w)], idx_v)

        hb_bits = _HB.bit_length() - 1

        def xform(i, carry):
            v = idx_v[pl.ds(i * L, L)]
            hi = jax.lax.shift_left(
                jax.lax.shift_right_logical(v, hb_bits + 1), hb_bits + 1
            )
            lo = jax.lax.shift_left(jax.lax.bitwise_and(v, _HB - 1), 1)
            b1 = jax.lax.bitwise_and(
                jax.lax.shift_right_logical(v, hb_bits), 1
            )
            idx2_v[pl.ds(i * L, L)] = hi + lo + b1
            return carry

        lax.fori_loop(0, per_w // L, xform, None)

        gsem = (g0, g1)
        osem = (o0, o1)
        gd = {0: [], 1: []}
        od = {0: None, 1: None}

        def fire(g):
            b = g % 2
            gd[b] = [
                pltpu.async_copy(
                    tab_hbm.at[idx2_v.at[pl.ds(g * C + k * G, G)]],
                    rows_v.at[b, pl.ds(k * G, G)],
                    gsem[b],
                )
                for k in range(K)
            ]

        fire(0)
        for g in range(n_chunks):
            b = g % 2
            if g + 1 < n_chunks:
                if od[1 - b] is not None:
                    od[1 - b].wait()
                    od[1 - b] = None
                fire(g + 1)
            for d in gd[b]:
                d.wait()
            od[b] = pltpu.async_copy(
                rows_v.at[b], out_hbm.at[pl.ds(base + g * C, C)], osem[b]
            )
        for b in (0, 1):
            if od[b] is not None:
                od[b].wait()

    out = emb(idx_flat, t64)
    return out.reshape(B, S, D)


# speedup vs baseline: 2.2537x; 1.0340x over previous
import functools

import jax
import jax.numpy as jnp
from jax import lax
from jax.experimental import pallas as pl
from jax.experimental.pallas import tpu as pltpu
from jax.experimental.pallas import tpu_sc as plsc


_HB = 16384


def _transpose_pack_body(xe_ref, xo_ref, y_ref):
    d = xe_ref.shape[0]
    y_ref[:, 0:d] = jnp.swapaxes(xe_ref[...], 0, 1)
    y_ref[:, d : 2 * d] = jnp.swapaxes(xo_ref[...], 0, 1)


def _transpose_pack(table_t):
    D, V = table_t.shape
    grid = (V + 2 * _HB - 1) // (2 * _HB)
    return pl.pallas_call(
        _transpose_pack_body,
        grid=(grid,),
        in_specs=[
            pl.BlockSpec((D, _HB), lambda i: (0, 2 * i)),
            pl.BlockSpec(
                (D, _HB),
                lambda i: (0, jnp.minimum(2 * i + 1, (V + _HB - 1) // _HB - 1)),
            ),
        ],
        out_specs=pl.BlockSpec((_HB, 2 * D), lambda i: (i, 0)),
        out_shape=jax.ShapeDtypeStruct((grid * _HB, 2 * D), jnp.float32),
    )(table_t, table_t)


def kernel(inputs, table):
    B, S = inputs.shape
    V, D = table.shape
    N = B * S

    tpack = _transpose_pack(jnp.swapaxes(table, 0, 1))
    t64 = tpack.reshape(2 * tpack.shape[0], D)

    info = plsc.get_sparse_core_info()
    NC, NS, L = info.num_cores, info.num_subcores, info.num_lanes
    NW = NC * NS
    per_w = N // NW
    G = 128
    K = 5
    C = G * K
    n_chunks = per_w // C

    idx_flat = inputs.reshape(N)
    mesh = plsc.VectorSubcoreMesh(core_axis_name="c", subcore_axis_name="s")

    @functools.partial(
        pl.kernel,
        mesh=mesh,
        compiler_params=pltpu.CompilerParams(
            use_tc_tiling_on_sc=False, needs_layout_passes=False
        ),
        out_type=jax.ShapeDtypeStruct((N, D), jnp.float32),
        scratch_types=[
            pltpu.VMEM((per_w,), jnp.int32),
            pltpu.VMEM((per_w,), jnp.int32),
            pltpu.VMEM((2, C, D), jnp.float32),
            pltpu.SemaphoreType.DMA,
            pltpu.SemaphoreType.DMA,
            pltpu.SemaphoreType.DMA,
            pltpu.SemaphoreType.DMA,
        ],
    )
    def emb(idx_hbm, tab_hbm, out_hbm, idx_v, idx2_v, rows_v, g0, g1, o0, o1):
        wid = lax.axis_index("s") * NC + lax.axis_index("c")
        base = wid * per_w
        pltpu.sync_copy(idx_hbm.at[pl.ds(base, per_w)], idx_v)

        hb_bits = _HB.bit_length() - 1

        def xform(i, carry):
            v = idx_v[pl.ds(i * L, L)]
            hi = jax.lax.shift_left(
                jax.lax.shift_right_logical(v, hb_bits + 1), hb_bits + 1
            )
            lo = jax.lax.shift_left(jax.lax.bitwise_and(v, _HB - 1), 1)
            b1 = jax.lax.bitwise_and(
                jax.lax.shift_right_logical(v, hb_bits), 1
            )
            idx2_v[pl.ds(i * L, L)] = hi + lo + b1
            return carry

        lax.fori_loop(0, per_w // L, xform, None)

        gsem = (g0, g1)
        osem = (o0, o1)
        gd = {0: [], 1: []}
        od = {0: None, 1: None}

        def fire(g):
            b = g % 2
            gd[b] = [
                pltpu.async_copy(
                    tab_hbm.at[idx2_v.at[pl.ds(g * C + k * G, G)]],
                    rows_v.at[b, pl.ds(k * G, G)],
                    gsem[b],
                )
                for k in range(K)
            ]

        fire(0)
        for g in range(n_chunks):
            b = g % 2
            if g + 1 < n_chunks:
                if od[1 - b] is not None:
                    od[1 - b].wait()
                    od[1 - b] = None
                fire(g + 1)
            for d in gd[b]:
                d.wait()
            od[b] = pltpu.async_copy(
                rows_v.at[b], out_hbm.at[pl.ds(base + g * C, C)], osem[b]
            )
        for b in (0, 1):
            if od[b] is not None:
                od[b].wait()

    out = emb(idx_flat, t64)
    return out.reshape(B, S, D)


# speedup vs baseline: 2.8900x; 1.2824x over previous
import functools

import jax
import jax.numpy as jnp
from jax import lax
from jax.experimental import pallas as pl
from jax.experimental.pallas import tpu as pltpu
from jax.experimental.pallas import tpu_sc as plsc


_HB = 16384


def _transpose_pack_body(xe_ref, xo_ref, y_ref):
    d = xe_ref.shape[0]
    y_ref[:, 0:d] = jnp.swapaxes(xe_ref[...], 0, 1)
    y_ref[:, d : 2 * d] = jnp.swapaxes(xo_ref[...], 0, 1)


def _transpose_pack(table_t):
    D, V = table_t.shape
    grid = (V + 2 * _HB - 1) // (2 * _HB)
    return pl.pallas_call(
        _transpose_pack_body,
        grid=(grid,),
        in_specs=[
            pl.BlockSpec((D, _HB), lambda i: (0, 2 * i)),
            pl.BlockSpec(
                (D, _HB),
                lambda i: (0, jnp.minimum(2 * i + 1, (V + _HB - 1) // _HB - 1)),
            ),
        ],
        out_specs=pl.BlockSpec((_HB, 2 * D), lambda i: (i, 0)),
        out_shape=jax.ShapeDtypeStruct((grid * _HB, 2 * D), jnp.float32),
    )(table_t, table_t)


_BB = 256


def _epilogue_body(x_ref, y_ref):
    rows_per_b = x_ref.shape[0] // _BB
    x2 = x_ref[...].reshape(_BB, rows_per_b * x_ref.shape[1])
    y_ref[...] = jnp.swapaxes(x2, 0, 1)


def _epilogue(sc_out, B, S, D):
    half = S * D // (2 * D)
    grid = B // _BB
    return pl.pallas_call(
        _epilogue_body,
        grid=(grid,),
        in_specs=[pl.BlockSpec((_BB * half, 2 * D), lambda i: (i, 0))],
        out_specs=pl.BlockSpec((S * D, _BB), lambda i: (0, i)),
        out_shape=jax.ShapeDtypeStruct((S * D, B), jnp.float32),
    )(sc_out)


def kernel(inputs, table):
    B, S = inputs.shape
    V, D = table.shape
    N = B * S

    tpack = _transpose_pack(jnp.swapaxes(table, 0, 1))
    t64 = tpack.reshape(2 * tpack.shape[0], D)

    info = plsc.get_sparse_core_info()
    NC, NS, L = info.num_cores, info.num_subcores, info.num_lanes
    NW = NC * NS
    per_w = N // NW
    G = 128
    K = 5
    C = G * K
    n_chunks = per_w // C

    idx_flat = inputs.reshape(N)
    mesh = plsc.VectorSubcoreMesh(core_axis_name="c", subcore_axis_name="s")

    @functools.partial(
        pl.kernel,
        mesh=mesh,
        compiler_params=pltpu.CompilerParams(
            use_tc_tiling_on_sc=False, needs_layout_passes=False
        ),
        out_type=jax.ShapeDtypeStruct((N, D), jnp.float32),
        scratch_types=[
            pltpu.VMEM((per_w,), jnp.int32),
            pltpu.VMEM((per_w,), jnp.int32),
            pltpu.VMEM((2, C, D), jnp.float32),
            pltpu.SemaphoreType.DMA,
            pltpu.SemaphoreType.DMA,
            pltpu.SemaphoreType.DMA,
            pltpu.SemaphoreType.DMA,
        ],
    )
    def emb(idx_hbm, tab_hbm, out_hbm, idx_v, idx2_v, rows_v, g0, g1, o0, o1):
        wid = lax.axis_index("s") * NC + lax.axis_index("c")
        base = wid * per_w
        pltpu.sync_copy(idx_hbm.at[pl.ds(base, per_w)], idx_v)

        hb_bits = _HB.bit_length() - 1

        def xform(i, carry):
            v = idx_v[pl.ds(i * L, L)]
            hi = jax.lax.shift_left(
                jax.lax.shift_right_logical(v, hb_bits + 1), hb_bits + 1
            )
            lo = jax.lax.shift_left(jax.lax.bitwise_and(v, _HB - 1), 1)
            b1 = jax.lax.bitwise_and(
                jax.lax.shift_right_logical(v, hb_bits), 1
            )
            idx2_v[pl.ds(i * L, L)] = hi + lo + b1
            return carry

        lax.fori_loop(0, per_w // L, xform, None)

        gsem = (g0, g1)
        osem = (o0, o1)
        gd = {0: [], 1: []}
        od = {0: None, 1: None}

        def fire(g):
            b = g % 2
            gd[b] = [
                pltpu.async_copy(
                    tab_hbm.at[idx2_v.at[pl.ds(g * C + k * G, G)]],
                    rows_v.at[b, pl.ds(k * G, G)],
                    gsem[b],
                )
                for k in range(K)
            ]

        fire(0)
        for g in range(n_chunks):
            b = g % 2
            if g + 1 < n_chunks:
                if od[1 - b] is not None:
                    od[1 - b].wait()
                    od[1 - b] = None
                fire(g + 1)
            for d in gd[b]:
                d.wait()
            od[b] = pltpu.async_copy(
                rows_v.at[b], out_hbm.at[pl.ds(base + g * C, C)], osem[b]
            )
        for b in (0, 1):
            if od[b] is not None:
                od[b].wait()

    sc_out = emb(idx_flat, t64)
    p2 = _epilogue(sc_out.reshape(N // 2, 2 * D), B, S, D)
    return jnp.transpose(p2.reshape(S, D, B), (2, 0, 1))
